# baseline (speedup 1.0000x reference)
import functools

import jax
import jax.numpy as jnp
from jax import lax
from jax.experimental import pallas as pl
from jax.experimental.pallas import tpu as pltpu
from jax.experimental.pallas import tpu_sc as plsc

N, E, D, DE, H = 10000, 320000, 128, 16, 32

NC, NS = 2, 16
NW = NC * NS
CB = 128
JPW = 80
E_PAD = NW * JPW * CB
EC = E // CB
N_ACC = 10112
RPT = N_ACC // NS
AW = H + 16
L = 16


ARS = H + 1


def _nodeproj_body(x_ref, wa_ref, wb_ref, b1_ref, ps_ref, pd_ref):
    x = x_ref[...]
    ps_ref[...] = jnp.dot(x, wa_ref[...], preferred_element_type=jnp.float32)
    pd_ref[...] = jnp.dot(x, wb_ref[...], preferred_element_type=jnp.float32) + b1_ref[...]


def _node_projections(x_pad, w1aT, w1bT, b1):
    blk = N_ACC // 4
    return pl.pallas_call(
        _nodeproj_body,
        grid=(4,),
        in_specs=[
            pl.BlockSpec((blk, D), lambda i: (i, 0)),
            pl.BlockSpec((D, H), lambda i: (0, 0)),
            pl.BlockSpec((D, H), lambda i: (0, 0)),
            pl.BlockSpec((H,), lambda i: (0,)),
        ],
        out_specs=[
            pl.BlockSpec((blk, H), lambda i: (i, 0)),
            pl.BlockSpec((blk, H), lambda i: (i, 0)),
        ],
        out_shape=[
            jax.ShapeDtypeStruct((N_ACC, H), jnp.float32),
            jax.ShapeDtypeStruct((N_ACC, H), jnp.float32),
        ],
    )(x_pad, w1aT, w1bT, b1)


def _attrproj_body(at_ref, w_ref, out_ref):
    at = at_ref[...]
    w = w_ref[...]
    acc = at[0][None, :] * w[:, 0][:, None]
    acc2 = at[1][None, :] * w[:, 1][:, None]
    for k in range(2, DE, 2):
        acc = acc + at[k][None, :] * w[:, k][:, None]
        acc2 = acc2 + at[k + 1][None, :] * w[:, k + 1][:, None]
    out_ref[...] = acc + acc2


def _attr_projection(attr_t, w1c):
    blk = 32000
    return pl.pallas_call(
        _attrproj_body,
        grid=(E // blk,),
        in_specs=[
            pl.BlockSpec((DE, blk), lambda i: (0, i)),
            pl.BlockSpec((H, DE), lambda i: (0, 0)),
        ],
        out_specs=pl.BlockSpec((H, blk), lambda i: (0, i)),
        out_shape=jax.ShapeDtypeStruct((H, E), jnp.float32),
    )(attr_t, w1c)


def _edge_body(ps_hbm, pd_hbm, attr_hbm, src_hbm, dst_hbm, zacc_hbm,
               hinit_hbm,
               acc_out,
               idx_s, idx_d, a_rm, rows_s0, rows_d0, at_v0, h_v0, rows_s1,
               rows_d1, at_v1, h_v1, acc_sh,
               gs0, gd0, ga0, ss0, gs1, gd1, ga1, ss1):
    cid = lax.axis_index("c")
    sid = lax.axis_index("s")
    wid = sid * NC + cid
    base = sid * RPT

    pltpu.sync_copy(zacc_hbm, acc_sh.at[pl.ds(base, RPT)])
    pltpu.sync_copy(hinit_hbm, h_v0)
    pltpu.sync_copy(hinit_hbm, h_v1)
    pltpu.sync_copy(src_hbm.at[wid], idx_s)
    pltpu.sync_copy(dst_hbm.at[wid], idx_d)
    plsc.subcore_barrier()

    bufs = ((rows_s0, rows_d0, at_v0, h_v0, gs0, gd0, ga0, ss0),
            (rows_s1, rows_d1, at_v1, h_v1, gs1, gd1, ga1, ss1))

    def start_gathers(jj, b):
        rs, rd, av = bufs[b][0], bufs[b][1], bufs[b][2]
        pltpu.async_copy(ps_hbm.at[idx_s.at[jj]], rs, bufs[b][4])
        pltpu.async_copy(pd_hbm.at[idx_d.at[jj]], rd, bufs[b][5])

        @pl.when(wid * JPW + jj < EC)
        def _():
            pltpu.async_copy(attr_hbm.at[:, wid * JPW + jj], av, bufs[b][6])

    start_gathers(0, 0)
    start_gathers(1, 1)

    def pair_body(i, carry):
        for b in range(2):
            jj = 2 * i + b
            rs, rd, av, hv, gs, gd, ga, ss = bufs[b]
            pltpu.make_async_copy(ps_hbm.at[idx_s.at[jj]], rs, gs).wait()
            pltpu.make_async_copy(pd_hbm.at[idx_d.at[jj]], rd, gd).wait()

            @pl.when(wid * JPW + jj < EC)
            def _():
                pltpu.make_async_copy(
                    attr_hbm.at[:, wid * JPW + jj], av, ga).wait()

            @pl.when(i > 0)
            def _():
                pltpu.make_async_copy(hv, acc_sh.at[idx_d.at[jj]], ss).wait()

            iota16 = jax.lax.iota(jnp.int32, L)

            def tr_body(g, carry2):
                ridx = (iota16 + g * L) * ARS
                for c in range(H):
                    v = av[c, pl.ds(g * L, L)]
                    plsc.store_scatter(a_rm, [ridx + c], v)
                return carry2

            lax.fori_loop(0, CB // L, tr_body, 0)

            def row_body(r, carry2):
                a0 = a_rm[pl.ds(r * ARS, L)]
                a1 = a_rm[pl.ds(r * ARS + L, L)]
                h0 = jnp.maximum(rs[r, pl.ds(0, L)] + rd[r, pl.ds(0, L)] + a0, 0.0)
                h1 = jnp.maximum(rs[r, pl.ds(L, L)] + rd[r, pl.ds(L, L)] + a1, 0.0)
                hv[r, pl.ds(0, H)] = plsc.pack(h0, h1,
                                               format=plsc.PackFormat.INTERLEAVED)
                return carry2

            lax.fori_loop(0, CB, row_body, 0, unroll=2)

            pltpu.async_copy(hv, acc_sh.at[idx_d.at[jj]], ss, add=True)

            @pl.when(i < JPW // 2 - 1)
            def _():
                start_gathers(jj + 2, b)
        return carry

    lax.fori_loop(0, JPW // 2, pair_body, 0)

    pltpu.make_async_copy(h_v0, acc_sh.at[idx_d.at[JPW - 2]], ss0).wait()
    pltpu.make_async_copy(h_v1, acc_sh.at[idx_d.at[JPW - 1]], ss1).wait()
    plsc.subcore_barrier()
    pltpu.sync_copy(acc_sh.at[pl.ds(base, RPT)], acc_out.at[cid, pl.ds(base, RPT)])


_edge_kernel = pl.kernel(
    _edge_body,
    out_type=jax.ShapeDtypeStruct((NC, N_ACC, AW), jnp.bfloat16),
    mesh=plsc.VectorSubcoreMesh(core_axis_name="c", subcore_axis_name="s"),
    scratch_types=[
        pltpu.VMEM((JPW, CB), jnp.int32),
        pltpu.VMEM((JPW, CB), jnp.int32),
        pltpu.VMEM((CB * ARS,), jnp.float32),
        pltpu.VMEM((CB, H), jnp.float32),
        pltpu.VMEM((CB, H), jnp.float32),
        pltpu.VMEM((H, CB), jnp.float32),
        pltpu.VMEM((CB, AW), jnp.bfloat16),
        pltpu.VMEM((CB, H), jnp.float32),
        pltpu.VMEM((CB, H), jnp.float32),
        pltpu.VMEM((H, CB), jnp.float32),
        pltpu.VMEM((CB, AW), jnp.bfloat16),
        pltpu.VMEM_SHARED((N_ACC, AW), jnp.bfloat16),
        pltpu.SemaphoreType.DMA,
        pltpu.SemaphoreType.DMA,
        pltpu.SemaphoreType.DMA,
        pltpu.SemaphoreType.DMA,
        pltpu.SemaphoreType.DMA,
        pltpu.SemaphoreType.DMA,
        pltpu.SemaphoreType.DMA,
        pltpu.SemaphoreType.DMA,
    ],
    compiler_params=pltpu.CompilerParams(use_tc_tiling_on_sc=False,
                                         needs_layout_passes=False),
)


def _post_body(x_ref, a0_ref, a1_ref, w2T_ref, b2_ref,
               w3aT_ref, w3bT_ref, b3_ref, out_ref):
    x = x_ref[...]
    acc = (a0_ref[0].astype(jnp.float32) + a1_ref[0].astype(jnp.float32))
    hs = acc[:, :H]
    cnt = acc[:, H:H + 1]
    agg = (jnp.dot(hs, w2T_ref[...], preferred_element_type=jnp.float32)
           + cnt * b2_ref[...]) / (cnt + 1e-6)
    xn = (jnp.dot(x, w3aT_ref[...], preferred_element_type=jnp.float32)
          + jnp.dot(agg, w3bT_ref[...], preferred_element_type=jnp.float32)
          + b3_ref[...])
    out_ref[...] = x + jnp.maximum(xn, 0.0)


def _post_update(x, accs, w2T, b2, w3aT, w3bT, b3):
    blk = 1000
    return pl.pallas_call(
        _post_body,
        grid=(N // blk,),
        in_specs=[
            pl.BlockSpec((blk, D), lambda i: (i, 0)),
            pl.BlockSpec((1, blk, AW), lambda i: (0, i, 0)),
            pl.BlockSpec((1, blk, AW), lambda i: (1, i, 0)),
            pl.BlockSpec((H, H), lambda i: (0, 0)),
            pl.BlockSpec((H,), lambda i: (0,)),
            pl.BlockSpec((D, D), lambda i: (0, 0)),
            pl.BlockSpec((H, D), lambda i: (0, 0)),
            pl.BlockSpec((D,), lambda i: (0,)),
        ],
        out_specs=pl.BlockSpec((blk, D), lambda i: (i, 0)),
        out_shape=jax.ShapeDtypeStruct((N, D), jnp.float32),
    )(x, accs, accs, w2T, b2, w3aT, w3bT, b3)


def kernel(x, edge_index, edge_attr, W1, b1, W2, b2, W3, b3):
    w1aT = W1[:, :D].T
    w1bT = W1[:, D:2 * D].T
    w1cT = W1[:, 2 * D:].T
    w2T = W2.T
    w3aT = W3[:, :D].T
    w3bT = W3[:, D:].T

    x_pad = jnp.pad(x, ((0, N_ACC - N), (0, 0)))
    src = jnp.pad(edge_index[0], (0, E_PAD - E)).reshape(NW, JPW, CB)
    dst = jnp.pad(edge_index[1], (0, E_PAD - E), constant_values=N).reshape(NW, JPW, CB)

    ps, pd = _node_projections(x_pad, w1aT, w1bT, b1)

    zacc = jnp.zeros((RPT, AW), jnp.bfloat16)
    hinit = jnp.zeros((CB, AW), jnp.bfloat16).at[:, H].set(1.0)

    perm = jnp.array([(c % 2) * L + c // 2 for c in range(H)], jnp.int32)
    w2T = w2T[perm, :]

    at3 = _attr_projection(edge_attr.T, W1[:, 2 * D:]).reshape(H, EC, CB)
    accs = _edge_kernel(ps, pd, at3, src, dst, zacc, hinit)

    return _post_update(x, accs, w2T, b2, w3aT, w3bT, b3)


# speedup vs baseline: 1.3168x; 1.3168x over previous
import functools

import jax
import jax.numpy as jnp
from jax import lax
from jax.experimental import pallas as pl
from jax.experimental.pallas import tpu as pltpu
from jax.experimental.pallas import tpu_sc as plsc

N, E, D, DE, H = 10000, 320000, 128, 16, 32

NC, NS = 2, 16
NW = NC * NS
CB = 128
JPW = 80
E_PAD = NW * JPW * CB
EC = E // CB
N_ACC = 10112
RPT = N_ACC // NS
AW = H + 16
L = 16


ARS = H + 1


def _nodeproj_body(x_ref, wa_ref, wb_ref, b1_ref, ps_ref, pd_ref):
    x = x_ref[...]
    ps_ref[...] = jnp.dot(x, wa_ref[...], preferred_element_type=jnp.float32)
    pd_ref[...] = jnp.dot(x, wb_ref[...], preferred_element_type=jnp.float32) + b1_ref[...]


def _node_projections(x_pad, w1aT, w1bT, b1):
    blk = N_ACC // 4
    return pl.pallas_call(
        _nodeproj_body,
        grid=(4,),
        in_specs=[
            pl.BlockSpec((blk, D), lambda i: (i, 0)),
            pl.BlockSpec((D, H), lambda i: (0, 0)),
            pl.BlockSpec((D, H), lambda i: (0, 0)),
            pl.BlockSpec((H,), lambda i: (0,)),
        ],
        out_specs=[
            pl.BlockSpec((blk, H), lambda i: (i, 0)),
            pl.BlockSpec((blk, H), lambda i: (i, 0)),
        ],
        out_shape=[
            jax.ShapeDtypeStruct((N_ACC, H), jnp.float32),
            jax.ShapeDtypeStruct((N_ACC, H), jnp.float32),
        ],
    )(x_pad, w1aT, w1bT, b1)


def _attrproj_body(at_ref, w_ref, out_ref):
    out_ref[...] = jnp.dot(w_ref[...], at_ref[...],
                           preferred_element_type=jnp.float32)


def _attr_projection(attr_t, w1c):
    blk = 32000
    return pl.pallas_call(
        _attrproj_body,
        grid=(E // blk,),
        in_specs=[
            pl.BlockSpec((DE, blk), lambda i: (0, i)),
            pl.BlockSpec((H, DE), lambda i: (0, 0)),
        ],
        out_specs=pl.BlockSpec((H, blk), lambda i: (0, i)),
        out_shape=jax.ShapeDtypeStruct((H, E), jnp.float32),
    )(attr_t, w1c)


def _edge_body(ps_hbm, pd_hbm, attr_hbm, src_hbm, dst_hbm, zacc_hbm,
               hinit_hbm,
               acc_out,
               idx_s, idx_d, a_rm, rows_s0, rows_d0, at_v0, h_v0, rows_s1,
               rows_d1, at_v1, h_v1, acc_sh,
               gs0, gd0, ga0, ss0, gs1, gd1, ga1, ss1):
    cid = lax.axis_index("c")
    sid = lax.axis_index("s")
    wid = sid * NC + cid
    base = sid * RPT

    pltpu.sync_copy(zacc_hbm, acc_sh.at[pl.ds(base, RPT)])
    pltpu.sync_copy(hinit_hbm, h_v0)
    pltpu.sync_copy(hinit_hbm, h_v1)
    pltpu.sync_copy(src_hbm.at[wid], idx_s)
    pltpu.sync_copy(dst_hbm.at[wid], idx_d)
    plsc.subcore_barrier()

    bufs = ((rows_s0, rows_d0, at_v0, h_v0, gs0, gd0, ga0, ss0),
            (rows_s1, rows_d1, at_v1, h_v1, gs1, gd1, ga1, ss1))

    def start_gathers(jj, b):
        rs, rd, av = bufs[b][0], bufs[b][1], bufs[b][2]
        pltpu.async_copy(ps_hbm.at[idx_s.at[jj]], rs, bufs[b][4])
        pltpu.async_copy(pd_hbm.at[idx_d.at[jj]], rd, bufs[b][5])

        @pl.when(wid * JPW + jj < EC)
        def _():
            pltpu.async_copy(attr_hbm.at[:, wid * JPW + jj], av, bufs[b][6])

    start_gathers(0, 0)
    start_gathers(1, 1)

    def pair_body(i, carry):
        for b in range(2):
            jj = 2 * i + b
            rs, rd, av, hv, gs, gd, ga, ss = bufs[b]
            pltpu.make_async_copy(ps_hbm.at[idx_s.at[jj]], rs, gs).wait()
            pltpu.make_async_copy(pd_hbm.at[idx_d.at[jj]], rd, gd).wait()

            @pl.when(wid * JPW + jj < EC)
            def _():
                pltpu.make_async_copy(
                    attr_hbm.at[:, wid * JPW + jj], av, ga).wait()

            @pl.when(i > 0)
            def _():
                pltpu.make_async_copy(hv, acc_sh.at[idx_d.at[jj]], ss).wait()

            iota16 = jax.lax.iota(jnp.int32, L)

            def tr_body(g, carry2):
                ridx = (iota16 + g * L) * ARS
                for c in range(H):
                    v = av[c, pl.ds(g * L, L)]
                    plsc.store_scatter(a_rm, [ridx + c], v)
                return carry2

            lax.fori_loop(0, CB // L, tr_body, 0)

            def row_body(r, carry2):
                a0 = a_rm[pl.ds(r * ARS, L)]
                a1 = a_rm[pl.ds(r * ARS + L, L)]
                h0 = jnp.maximum(rs[r, pl.ds(0, L)] + rd[r, pl.ds(0, L)] + a0, 0.0)
                h1 = jnp.maximum(rs[r, pl.ds(L, L)] + rd[r, pl.ds(L, L)] + a1, 0.0)
                hv[r, pl.ds(0, H)] = plsc.pack(h0, h1,
                                               format=plsc.PackFormat.INTERLEAVED)
                return carry2

            lax.fori_loop(0, CB, row_body, 0, unroll=2)

            pltpu.async_copy(hv, acc_sh.at[idx_d.at[jj]], ss, add=True)

            @pl.when(i < JPW // 2 - 1)
            def _():
                start_gathers(jj + 2, b)
        return carry

    lax.fori_loop(0, JPW // 2, pair_body, 0)

    pltpu.make_async_copy(h_v0, acc_sh.at[idx_d.at[JPW - 2]], ss0).wait()
    pltpu.make_async_copy(h_v1, acc_sh.at[idx_d.at[JPW - 1]], ss1).wait()
    plsc.subcore_barrier()
    pltpu.sync_copy(acc_sh.at[pl.ds(base, RPT)], acc_out.at[cid, pl.ds(base, RPT)])


_edge_kernel = pl.kernel(
    _edge_body,
    out_type=jax.ShapeDtypeStruct((NC, N_ACC, AW), jnp.bfloat16),
    mesh=plsc.VectorSubcoreMesh(core_axis_name="c", subcore_axis_name="s"),
    scratch_types=[
        pltpu.VMEM((JPW, CB), jnp.int32),
        pltpu.VMEM((JPW, CB), jnp.int32),
        pltpu.VMEM((CB * ARS,), jnp.float32),
        pltpu.VMEM((CB, H), jnp.float32),
        pltpu.VMEM((CB, H), jnp.float32),
        pltpu.VMEM((H, CB), jnp.float32),
        pltpu.VMEM((CB, AW), jnp.bfloat16),
        pltpu.VMEM((CB, H), jnp.float32),
        pltpu.VMEM((CB, H), jnp.float32),
        pltpu.VMEM((H, CB), jnp.float32),
        pltpu.VMEM((CB, AW), jnp.bfloat16),
        pltpu.VMEM_SHARED((N_ACC, AW), jnp.bfloat16),
        pltpu.SemaphoreType.DMA,
        pltpu.SemaphoreType.DMA,
        pltpu.SemaphoreType.DMA,
        pltpu.SemaphoreType.DMA,
        pltpu.SemaphoreType.DMA,
        pltpu.SemaphoreType.DMA,
        pltpu.SemaphoreType.DMA,
        pltpu.SemaphoreType.DMA,
    ],
    compiler_params=pltpu.CompilerParams(use_tc_tiling_on_sc=False,
                                         needs_layout_passes=False),
)


def _post_body(x_ref, a0_ref, a1_ref, w2T_ref, b2_ref,
               w3aT_ref, w3bT_ref, b3_ref, out_ref):
    x = x_ref[...]
    acc = (a0_ref[0].astype(jnp.float32) + a1_ref[0].astype(jnp.float32))
    hs = acc[:, :H]
    cnt = acc[:, H:H + 1]
    agg = (jnp.dot(hs, w2T_ref[...], preferred_element_type=jnp.float32)
           + cnt * b2_ref[...]) / (cnt + 1e-6)
    xn = (jnp.dot(x, w3aT_ref[...], preferred_element_type=jnp.float32)
          + jnp.dot(agg, w3bT_ref[...], preferred_element_type=jnp.float32)
          + b3_ref[...])
    out_ref[...] = x + jnp.maximum(xn, 0.0)


def _post_update(x, accs, w2T, b2, w3aT, w3bT, b3):
    blk = 1000
    return pl.pallas_call(
        _post_body,
        grid=(N // blk,),
        in_specs=[
            pl.BlockSpec((blk, D), lambda i: (i, 0)),
            pl.BlockSpec((1, blk, AW), lambda i: (0, i, 0)),
            pl.BlockSpec((1, blk, AW), lambda i: (1, i, 0)),
            pl.BlockSpec((H, H), lambda i: (0, 0)),
            pl.BlockSpec((H,), lambda i: (0,)),
            pl.BlockSpec((D, D), lambda i: (0, 0)),
            pl.BlockSpec((H, D), lambda i: (0, 0)),
            pl.BlockSpec((D,), lambda i: (0,)),
        ],
        out_specs=pl.BlockSpec((blk, D), lambda i: (i, 0)),
        out_shape=jax.ShapeDtypeStruct((N, D), jnp.float32),
    )(x, accs, accs, w2T, b2, w3aT, w3bT, b3)


def kernel(x, edge_index, edge_attr, W1, b1, W2, b2, W3, b3):
    w1aT = W1[:, :D].T
    w1bT = W1[:, D:2 * D].T
    w1cT = W1[:, 2 * D:].T
    w2T = W2.T
    w3aT = W3[:, :D].T
    w3bT = W3[:, D:].T

    x_pad = jnp.pad(x, ((0, N_ACC - N), (0, 0)))
    src = jnp.pad(edge_index[0], (0, E_PAD - E)).reshape(NW, JPW, CB)
    dst = jnp.pad(edge_index[1], (0, E_PAD - E), constant_values=N).reshape(NW, JPW, CB)

    ps, pd = _node_projections(x_pad, w1aT, w1bT, b1)

    zacc = jnp.zeros((RPT, AW), jnp.bfloat16)
    hinit = jnp.zeros((CB, AW), jnp.bfloat16).at[:, H].set(1.0)

    perm = jnp.array([(c % 2) * L + c // 2 for c in range(H)], jnp.int32)
    w2T = w2T[perm, :]

    at3 = _attr_projection(edge_attr.T, W1[:, 2 * D:]).reshape(H, EC, CB)
    accs = _edge_kernel(ps, pd, at3, src, dst, zacc, hinit)

    return _post_update(x, accs, w2T, b2, w3aT, w3bT, b3)
